# baseline (speedup 1.0000x reference)
import functools

import jax
import jax.numpy as jnp
from jax import lax
from jax.experimental import pallas as pl
from jax.experimental.pallas import tpu as pltpu
from jax.experimental.pallas import tpu_sc as plsc

NC, NS, L = 2, 16, 16
NW = NC * NS
N = 10000
NP = 10240
ZROW = N
TRASH = N + 16
E = 320000
EPT = 10240
NG = EPT // L
RPT = NP // NS
RTC = 1024
GRID = NP // RTC
DMASK = 16383

_mesh = plsc.VectorSubcoreMesh(core_axis_name="c", subcore_axis_name="s")


def _iota16():
    return lax.iota(jnp.int32, L)


def _dup_flag(flat, ob, j):
    iota = _iota16()
    base = j * L
    didx = flat[pl.ds(base, L)] & DMASK
    dup = didx != didx
    for sft in range(1, L):
        nxt = flat[pl.ds(base + sft, L)] & DMASK
        dup = dup | ((didx == nxt) & (iota < L - sft))
    ob[pl.ds(0, L)] = jnp.where(dup, 1, 0)
    for sft in (8, 4, 2, 1):
        ob[pl.ds(0, L)] = ob[pl.ds(0, L)] | ob[pl.ds(sft, L)]
    return ob[pl.ds(0, L)][0], didx


@functools.partial(
    pl.kernel,
    out_type=jax.ShapeDtypeStruct((NW * NG * L,), jnp.int32),
    mesh=_mesh,
    scratch_types=[
        pltpu.VMEM((NG * L,), jnp.int32),
        pltpu.VMEM((NG * L,), jnp.int32),
    ],
)
def _sc_prep(src_h, dst_h, packed_h, sv, dv):
    c = lax.axis_index("c")
    s = lax.axis_index("s")
    t = s * NC + c

    off = t * NG * L
    pltpu.sync_copy(src_h.at[pl.ds(off, NG * L)], sv)
    pltpu.sync_copy(dst_h.at[pl.ds(off, NG * L)], dv)

    def _pack(j, _):
        s16 = sv[pl.ds(j * L, L)]
        d16 = dv[pl.ds(j * L, L)]
        eff = jnp.where(s16 == d16, ZROW, s16)
        sv[pl.ds(j * L, L)] = lax.shift_left(eff, 14) | d16
        return 0
    lax.fori_loop(0, NG, _pack, 0)
    pltpu.sync_copy(sv, packed_h.at[pl.ds(off, NG * L)])


@functools.partial(
    pl.kernel,
    out_type=jax.ShapeDtypeStruct((NC * NP, 128), jnp.float32),
    mesh=_mesh,
    scratch_types=[
        pltpu.VMEM((NG * L + 2 * L,), jnp.int32),
        pltpu.VMEM((32,), jnp.int32),
        pltpu.VMEM((L, 128), jnp.float32),
        pltpu.VMEM((L, 128), jnp.float32),
        pltpu.VMEM((L, 128), jnp.float32),
        pltpu.VMEM((L, 128), jnp.float32),
        pltpu.VMEM((L, 128), jnp.float32),
        pltpu.VMEM((L, 128), jnp.float32),
        pltpu.VMEM_SHARED((NP, 128), jnp.float32),
        pltpu.SemaphoreType.DMA,
        pltpu.SemaphoreType.DMA,
        pltpu.SemaphoreType.DMA,
        pltpu.SemaphoreType.DMA,
        pltpu.SemaphoreType.DMA,
    ],
)
def _sc_propagate(g_h, packed_h, part_h, pv1, ob, b0, b1, b2, b3, xb, zb, acc,
                  s0, s1, s2, s3, sx):
    c = lax.axis_index("c")
    s = lax.axis_index("s")
    t = s * NC + c
    iota = _iota16()
    bufs = (b0, b1, b2, b3)
    sems = (s0, s1, s2, s3)

    pltpu.sync_copy(packed_h.at[pl.ds(t * NG * L, NG * L)],
                    pv1.at[pl.ds(0, NG * L)])
    ob[pl.ds(L, L)] = jnp.zeros((L,), jnp.int32)
    pv1[pl.ds(NG * L, L)] = jnp.zeros((L,), jnp.int32) - 1
    pv1[pl.ds(NG * L + L, L)] = jnp.zeros((L,), jnp.int32) - 1

    def _fill(i, _):
        for q in range(128 // L):
            zb[i, pl.ds(q * L, L)] = jnp.zeros((L,), jnp.float32)
        return 0
    lax.fori_loop(0, L, _fill, 0)

    def _zero(kz, _):
        pltpu.sync_copy(zb, acc.at[s * RPT + kz * L + iota])
        return 0
    lax.fori_loop(0, RPT // L, _zero, 0)
    pltpu.async_copy(acc.at[s * RPT + iota], xb, sx).wait()
    plsc.subcore_barrier()

    def _sidx(j):
        return lax.shift_right_logical(pv1[pl.ds(j * L, L)], 14)

    for b in range(4):
        pltpu.async_copy(g_h.at[_sidx(b)], bufs[b], sems[b])

    def _body(i, _):
        for b in range(4):
            j = 4 * i + b
            buf = bufs[b]
            pltpu.make_async_copy(g_h.at[_sidx(j)], buf, sems[b]).wait()
            flag, didx = _dup_flag(pv1, ob, j)

            @pl.when(flag == 0)
            def _():
                pltpu.sync_copy(buf, acc.at[didx], add=True)

            @pl.when(flag != 0)
            def _():
                def _fb(e, _2):
                    d_e = pv1[pl.ds(j * L + e, L)][0] & DMASK
                    idx_e = jnp.where(iota == 0, d_e, TRASH + iota)
                    for q in range(128 // L):
                        xb[0, pl.ds(q * L, L)] = buf[e, pl.ds(q * L, L)]
                    pltpu.sync_copy(xb, acc.at[idx_e], add=True)
                    return 0
                lax.fori_loop(0, L, _fb, 0)

            @pl.when(j + 4 < NG)
            def _():
                pltpu.async_copy(g_h.at[_sidx(j + 4)], buf, sems[b])
        return 0
    lax.fori_loop(0, NG // 4, _body, 0)

    pltpu.async_copy(acc.at[s * RPT + iota], xb, sx).wait()
    plsc.subcore_barrier()

    def _wout(kz, _):
        base = s * RPT + kz * L
        pltpu.async_copy(acc.at[base + iota], xb, sx).wait()
        pltpu.sync_copy(xb, part_h.at[pl.ds(c * NP + base, L)])
        return 0
    lax.fori_loop(0, RPT // L, _wout, 0)


def _tc_prep_body(degp_ref, x_ref, dinv_ref, g_ref):
    i = pl.program_id(0)
    deg = degp_ref[0][:, 0:1] + degp_ref[1][:, 0:1]
    rows = i * RTC + lax.broadcasted_iota(jnp.int32, (RTC, 1), 0)
    dinv = jnp.where((deg > 0.0) & (rows < N), lax.rsqrt(deg), 0.0)
    dinvb = jnp.broadcast_to(dinv, (RTC, 128))
    dinv_ref[...] = dinvb
    g_ref[...] = dinvb * x_ref[...]


def _tc_prep(degp, x_pad):
    return pl.pallas_call(
        _tc_prep_body,
        grid=(GRID,),
        in_specs=[
            pl.BlockSpec((NC, RTC, 128), lambda i: (0, i, 0)),
            pl.BlockSpec((RTC, 128), lambda i: (i, 0)),
        ],
        out_specs=[
            pl.BlockSpec((RTC, 128), lambda i: (i, 0)),
            pl.BlockSpec((RTC, 128), lambda i: (i, 0)),
        ],
        out_shape=[
            jax.ShapeDtypeStruct((NP, 128), jnp.float32),
            jax.ShapeDtypeStruct((NP, 128), jnp.float32),
        ],
    )(degp, x_pad)


def _tc_chain_body(part_ref, dinv_ref, g_ref):
    d = dinv_ref[...]
    g_ref[...] = -(d * d) * (part_ref[0] + part_ref[1])


def _tc_chain(part, dinvb):
    return pl.pallas_call(
        _tc_chain_body,
        grid=(GRID,),
        in_specs=[
            pl.BlockSpec((NC, RTC, 128), lambda i: (0, i, 0)),
            pl.BlockSpec((RTC, 128), lambda i: (i, 0)),
        ],
        out_specs=pl.BlockSpec((RTC, 128), lambda i: (i, 0)),
        out_shape=jax.ShapeDtypeStruct((NP, 128), jnp.float32),
    )(part, dinvb)


def _tc_layer1_body(x_ref, s1_ref, s2_ref, dinv_ref, w_ref, b_ref, h_ref, g_ref):
    d = dinv_ref[...]
    x = x_ref[...]
    t1 = -d * (s1_ref[0] + s1_ref[1])
    t2 = -2.0 * d * (s2_ref[0] + s2_ref[1]) - x
    out = jnp.dot(x, w_ref[0], preferred_element_type=jnp.float32)
    out = out + jnp.dot(t1, w_ref[1], preferred_element_type=jnp.float32)
    out = out + jnp.dot(t2, w_ref[2], preferred_element_type=jnp.float32)
    out = out + b_ref[...]
    h = jnp.maximum(out, 0.0)
    h_ref[...] = h
    g_ref[...] = d * h


def _tc_layer1(x_pad, s1, s2, dinvb, W1, b1):
    return pl.pallas_call(
        _tc_layer1_body,
        grid=(GRID,),
        in_specs=[
            pl.BlockSpec((RTC, 128), lambda i: (i, 0)),
            pl.BlockSpec((NC, RTC, 128), lambda i: (0, i, 0)),
            pl.BlockSpec((NC, RTC, 128), lambda i: (0, i, 0)),
            pl.BlockSpec((RTC, 128), lambda i: (i, 0)),
            pl.BlockSpec((3, 128, 128), lambda i: (0, 0, 0)),
            pl.BlockSpec((1, 128), lambda i: (0, 0)),
        ],
        out_specs=[
            pl.BlockSpec((RTC, 128), lambda i: (i, 0)),
            pl.BlockSpec((RTC, 128), lambda i: (i, 0)),
        ],
        out_shape=[
            jax.ShapeDtypeStruct((NP, 128), jnp.float32),
            jax.ShapeDtypeStruct((NP, 128), jnp.float32),
        ],
    )(x_pad, s1, s2, dinvb, W1, b1)


def _tc_layer2_body(h_ref, s3_ref, s4_ref, dinv_ref, w_ref, b_ref, o_ref):
    d = dinv_ref[...]
    h = h_ref[...]
    t1 = -d * (s3_ref[0] + s3_ref[1])
    t2 = -2.0 * d * (s4_ref[0] + s4_ref[1]) - h
    out = jnp.dot(h, w_ref[0], preferred_element_type=jnp.float32)
    out = out + jnp.dot(t1, w_ref[1], preferred_element_type=jnp.float32)
    out = out + jnp.dot(t2, w_ref[2], preferred_element_type=jnp.float32)
    o_ref[...] = out + b_ref[...]


def _tc_layer2(h_pad, s3, s4, dinvb, W2, b2):
    return pl.pallas_call(
        _tc_layer2_body,
        grid=(GRID,),
        in_specs=[
            pl.BlockSpec((RTC, 128), lambda i: (i, 0)),
            pl.BlockSpec((NC, RTC, 128), lambda i: (0, i, 0)),
            pl.BlockSpec((NC, RTC, 128), lambda i: (0, i, 0)),
            pl.BlockSpec((RTC, 128), lambda i: (i, 0)),
            pl.BlockSpec((3, 128, 64), lambda i: (0, 0, 0)),
            pl.BlockSpec((1, 64), lambda i: (0, 0)),
        ],
        out_specs=pl.BlockSpec((RTC, 64), lambda i: (i, 0)),
        out_shape=jax.ShapeDtypeStruct((NP, 64), jnp.float32),
    )(h_pad, s3, s4, dinvb, W2, b2)


def kernel(x, edge_index, W1, b1, W2, b2):
    src = edge_index[0].astype(jnp.int32)
    dst = edge_index[1].astype(jnp.int32)
    padlen = NW * EPT - E
    pidx = jnp.arange(padlen, dtype=jnp.int32) % L
    src3 = jnp.concatenate([src, N + 16 + pidx])
    dst3 = jnp.concatenate([dst, N + 32 + pidx])
    x_pad = jnp.pad(x, ((0, NP - N), (0, 0)))

    packed1 = _sc_prep(src3, dst3)
    eff1 = lax.shift_right_logical(packed1, 14)
    g_deg = jnp.zeros((NP, 128), jnp.float32).at[0].set(1.0)
    degp = _sc_propagate(g_deg, eff1).reshape(NC, NP, 128)
    dinvb, g1 = _tc_prep(degp, x_pad)

    def _prop(g, dep):
        pk = packed1 + (dep.ravel()[0] * 0.0).astype(jnp.int32)
        return _sc_propagate(g, pk).reshape(NC, NP, 128)

    s1 = _prop(g1, g1)
    g2 = _tc_chain(s1, dinvb)
    s2 = _prop(g2, s1)
    h_pad, g3 = _tc_layer1(x_pad, s1, s2, dinvb, W1, b1.reshape(1, 128))

    s3 = _prop(g3, s2)
    g4 = _tc_chain(s3, dinvb)
    s4 = _prop(g4, s3)
    out = _tc_layer2(h_pad, s3, s4, dinvb, W2, b2.reshape(1, 64))
    return out[:N]
